# baseline (speedup 1.0000x reference)
import functools

import jax
import jax.numpy as jnp
from jax import lax
from jax.experimental import pallas as pl
from jax.experimental.pallas import tpu as pltpu
from jax.experimental.pallas import tpu_sc as plsc

N = 10000
E = 320000
D = 128
OUT = 128

NC = 2
NS = 16
HD = D // NC
B = 128
NCH = 160
EPT = NCH * B
E_PAD = NS * EPT
N_PAD = 10240
N_DUMMY = N + 16
RPT = N_PAD // NS
CW = 8


def _sc_scatter_body(xlo_hbm, xhi_hbm, esrc_hbm, etgt_hbm,
                     zacc_hbm, zcnt_hbm, ones_hbm,
                     sums_out, cnts_out,
                     src_idx, tgt_idx, rows0, rows1, ones_v, acc, cnt,
                     gsem0, gsem1):
    c = lax.axis_index("c")
    s = lax.axis_index("s")

    pltpu.sync_copy(zacc_hbm.at[pl.ds(s * RPT, RPT)], acc.at[pl.ds(s * RPT, RPT)])
    pltpu.sync_copy(zcnt_hbm.at[pl.ds(s * RPT, RPT)], cnt.at[pl.ds(s * RPT, RPT)])

    pltpu.sync_copy(esrc_hbm.at[s], src_idx)
    pltpu.sync_copy(etgt_hbm.at[s], tgt_idx)
    pltpu.sync_copy(ones_hbm, ones_v)

    plsc.subcore_barrier()

    def start_gather(i, rows, gsem):
        @pl.when(c == 0)
        def _():
            pltpu.async_copy(xlo_hbm.at[src_idx.at[i]], rows, gsem)

        @pl.when(c == 1)
        def _():
            pltpu.async_copy(xhi_hbm.at[src_idx.at[i]], rows, gsem)

    def finish_chunk(i, rows, gsem):
        pltpu.make_async_copy(xlo_hbm.at[src_idx.at[i]], rows, gsem).wait()
        pltpu.sync_copy(rows, acc.at[tgt_idx.at[i]], add=True)

        @pl.when(lax.rem(i, 2) == c)
        def _():
            pltpu.sync_copy(ones_v, cnt.at[tgt_idx.at[i]], add=True)

    start_gather(0, rows0, gsem0)
    start_gather(1, rows1, gsem1)

    def chunk_pair(j, carry):
        i = 2 * j
        finish_chunk(i, rows0, gsem0)

        @pl.when(i + 2 < NCH)
        def _():
            start_gather(i + 2, rows0, gsem0)

        finish_chunk(i + 1, rows1, gsem1)

        @pl.when(i + 3 < NCH)
        def _():
            start_gather(i + 3, rows1, gsem1)

        return carry

    lax.fori_loop(0, NCH // 2, chunk_pair, 0)

    plsc.subcore_barrier()

    pltpu.sync_copy(acc.at[pl.ds(s * RPT, RPT)], sums_out.at[c, pl.ds(s * RPT, RPT)])
    pltpu.sync_copy(cnt.at[pl.ds(s * RPT, RPT)], cnts_out.at[c, pl.ds(s * RPT, RPT)])


_sc_scatter = functools.partial(
    pl.kernel,
    out_type=(
        jax.ShapeDtypeStruct((NC, N_PAD, HD), jnp.float32),
        jax.ShapeDtypeStruct((NC, N_PAD, CW), jnp.float32),
    ),
    mesh=plsc.VectorSubcoreMesh(
        core_axis_name="c", subcore_axis_name="s",
        num_cores=NC, num_subcores=NS),
    compiler_params=pltpu.CompilerParams(use_tc_tiling_on_sc=False),
    scratch_types=[
        pltpu.VMEM((NCH, B), jnp.int32),
        pltpu.VMEM((NCH, B), jnp.int32),
        pltpu.VMEM((B, HD), jnp.float32),
        pltpu.VMEM((B, HD), jnp.float32),
        pltpu.VMEM((B, CW), jnp.float32),
        pltpu.VMEM_SHARED((N_PAD, HD), jnp.float32),
        pltpu.VMEM_SHARED((N_PAD, CW), jnp.float32),
        pltpu.SemaphoreType.DMA,
        pltpu.SemaphoreType.DMA,
    ],
)(_sc_scatter_body)


def _combine_body(x_ref, s_ref, cp_ref, ws_ref, wt_ref, o_ref):
    cnt = cp_ref[0, :, 0:1] + cp_ref[1, :, 0:1]
    sums = jnp.concatenate([s_ref[0], s_ref[1]], axis=1)
    mean = sums / jnp.maximum(cnt, 1.0)
    o_ref[...] = (
        jnp.dot(x_ref[...], wt_ref[...], preferred_element_type=jnp.float32)
        + jnp.dot(mean, ws_ref[...], preferred_element_type=jnp.float32)
    )


BN = 2000


def _combine(x, sums, cnts_p, W_src, W_tgt):
    grid = (N // BN,)
    return pl.pallas_call(
        _combine_body,
        grid=grid,
        in_specs=[
            pl.BlockSpec((BN, D), lambda i: (i, 0)),
            pl.BlockSpec((2, BN, HD), lambda i: (0, i, 0)),
            pl.BlockSpec((2, BN, CW), lambda i: (0, i, 0)),
            pl.BlockSpec((D, OUT), lambda i: (0, 0)),
            pl.BlockSpec((D, OUT), lambda i: (0, 0)),
        ],
        out_specs=pl.BlockSpec((BN, OUT), lambda i: (i, 0)),
        out_shape=jax.ShapeDtypeStruct((N, OUT), jnp.float32),
    )(x, sums, cnts_p, W_src, W_tgt)


def kernel(x, edge_index, W_src, W_tgt):
    xlo = x[:, :HD]
    xhi = x[:, HD:]
    pad_src = jnp.zeros((E_PAD - E,), jnp.int32)
    pad_tgt = N + jnp.arange(E_PAD - E, dtype=jnp.int32) % (N_PAD - N)
    esrc = jnp.concatenate([edge_index[0], pad_src]).reshape(NS, NCH, B)
    etgt = jnp.concatenate([edge_index[1], pad_tgt]).reshape(NS, NCH, B)
    zacc = jnp.zeros((N_PAD, HD), jnp.float32)
    zcnt = jnp.zeros((N_PAD, CW), jnp.float32)
    ones = jnp.ones((B, CW), jnp.float32)
    sums, cnts_p = _sc_scatter(xlo, xhi, esrc, etgt, zacc, zcnt, ones)
    return _combine(x, sums, cnts_p, W_src, W_tgt)


# speedup vs baseline: 1.7915x; 1.7915x over previous
import functools

import jax
import jax.numpy as jnp
from jax import lax
from jax.experimental import pallas as pl
from jax.experimental.pallas import tpu as pltpu
from jax.experimental.pallas import tpu_sc as plsc

N = 10000
E = 320000
D = 128
OUT = 128

NC = 2
NS = 16
HD = D // NC
B = 125
EPT = E // NS
NCH = EPT // B
N_PAD = 10240
RPT = N_PAD // NS
CW = 8


def _sc_scatter_body(xlo_hbm, xhi_hbm, esrc_hbm, etgt_hbm,
                     zacc_hbm, zcnt_hbm, ones_hbm,
                     sums_out, cnts_out,
                     src_idx, tgt_idx, rows0, rows1, ones_v, acc, cnt,
                     gsem0, gsem1):
    c = lax.axis_index("c")
    s = lax.axis_index("s")

    pltpu.sync_copy(zacc_hbm.at[pl.ds(s * RPT, RPT)], acc.at[pl.ds(s * RPT, RPT)])
    pltpu.sync_copy(zcnt_hbm.at[pl.ds(s * RPT, RPT)], cnt.at[pl.ds(s * RPT, RPT)])

    pltpu.sync_copy(esrc_hbm.at[s], src_idx)
    pltpu.sync_copy(etgt_hbm.at[s], tgt_idx)
    pltpu.sync_copy(ones_hbm, ones_v)

    plsc.subcore_barrier()

    def start_gather(i, rows, gsem):
        @pl.when(c == 0)
        def _():
            pltpu.async_copy(xlo_hbm.at[src_idx.at[i]], rows, gsem)

        @pl.when(c == 1)
        def _():
            pltpu.async_copy(xhi_hbm.at[src_idx.at[i]], rows, gsem)

    def finish_chunk(i, rows, gsem):
        pltpu.make_async_copy(xlo_hbm.at[src_idx.at[i]], rows, gsem).wait()
        pltpu.sync_copy(rows, acc.at[tgt_idx.at[i]], add=True)

        @pl.when(lax.rem(i, 2) == c)
        def _():
            pltpu.sync_copy(ones_v, cnt.at[tgt_idx.at[i]], add=True)

    start_gather(0, rows0, gsem0)
    start_gather(1, rows1, gsem1)

    def chunk_pair(j, carry):
        i = 2 * j
        finish_chunk(i, rows0, gsem0)

        @pl.when(i + 2 < NCH)
        def _():
            start_gather(i + 2, rows0, gsem0)

        finish_chunk(i + 1, rows1, gsem1)

        @pl.when(i + 3 < NCH)
        def _():
            start_gather(i + 3, rows1, gsem1)

        return carry

    lax.fori_loop(0, NCH // 2, chunk_pair, 0)

    plsc.subcore_barrier()

    pltpu.sync_copy(acc.at[pl.ds(s * RPT, RPT)], sums_out.at[c, pl.ds(s * RPT, RPT)])
    pltpu.sync_copy(cnt.at[pl.ds(s * RPT, RPT)], cnts_out.at[c, pl.ds(s * RPT, RPT)])


_sc_scatter = functools.partial(
    pl.kernel,
    out_type=(
        jax.ShapeDtypeStruct((NC, N_PAD, HD), jnp.float32),
        jax.ShapeDtypeStruct((NC, N_PAD, CW), jnp.float32),
    ),
    mesh=plsc.VectorSubcoreMesh(
        core_axis_name="c", subcore_axis_name="s",
        num_cores=NC, num_subcores=NS),
    compiler_params=pltpu.CompilerParams(use_tc_tiling_on_sc=False),
    scratch_types=[
        pltpu.VMEM((NCH, B), jnp.int32),
        pltpu.VMEM((NCH, B), jnp.int32),
        pltpu.VMEM((B, HD), jnp.float32),
        pltpu.VMEM((B, HD), jnp.float32),
        pltpu.VMEM((B, CW), jnp.float32),
        pltpu.VMEM_SHARED((N_PAD, HD), jnp.float32),
        pltpu.VMEM_SHARED((N_PAD, CW), jnp.float32),
        pltpu.SemaphoreType.DMA,
        pltpu.SemaphoreType.DMA,
    ],
)(_sc_scatter_body)


def _combine_body(x_ref, s_ref, cp_ref, ws_ref, wt_ref, o_ref):
    cnt = cp_ref[0, :, 0:1] + cp_ref[1, :, 0:1]
    sums = jnp.concatenate([s_ref[0], s_ref[1]], axis=1)
    mean = sums / jnp.maximum(cnt, 1.0)
    o_ref[...] = (
        jnp.dot(x_ref[...], wt_ref[...], preferred_element_type=jnp.float32)
        + jnp.dot(mean, ws_ref[...], preferred_element_type=jnp.float32)
    )


BN = 2000


def _combine(x, sums, cnts_p, W_src, W_tgt):
    grid = (N // BN,)
    return pl.pallas_call(
        _combine_body,
        grid=grid,
        in_specs=[
            pl.BlockSpec((BN, D), lambda i: (i, 0)),
            pl.BlockSpec((2, BN, HD), lambda i: (0, i, 0)),
            pl.BlockSpec((2, BN, CW), lambda i: (0, i, 0)),
            pl.BlockSpec((D, OUT), lambda i: (0, 0)),
            pl.BlockSpec((D, OUT), lambda i: (0, 0)),
        ],
        out_specs=pl.BlockSpec((BN, OUT), lambda i: (i, 0)),
        out_shape=jax.ShapeDtypeStruct((N, OUT), jnp.float32),
    )(x, sums, cnts_p, W_src, W_tgt)


def kernel(x, edge_index, W_src, W_tgt):
    xlo = x[:, :HD]
    xhi = x[:, HD:]
    esrc = edge_index[0].reshape(NS, NCH, B)
    etgt = edge_index[1].reshape(NS, NCH, B)
    zacc = jnp.zeros((N_PAD, HD), jnp.float32)
    zcnt = jnp.zeros((N_PAD, CW), jnp.float32)
    ones = jnp.ones((B, CW), jnp.float32)
    sums, cnts_p = _sc_scatter(xlo, xhi, esrc, etgt, zacc, zcnt, ones)
    return _combine(x, sums, cnts_p, W_src, W_tgt)


# speedup vs baseline: 1.9999x; 1.1163x over previous
import functools

import jax
import jax.numpy as jnp
from jax import lax
from jax.experimental import pallas as pl
from jax.experimental.pallas import tpu as pltpu
from jax.experimental.pallas import tpu_sc as plsc

N = 10000
E = 320000
D = 128
OUT = 128

NC = 2
NS = 16
HD = D // NC
EPT = E // NS
B = 250
NCH = EPT // B
N_PAD = 10240
RPT = N_PAD // NS
CW = 8


def _sc_scatter_body(xlo_hbm, xhi_hbm, esrc_hbm, etgt_hbm,
                     zacc_hbm, zcnt_hbm, ones_hbm,
                     sums_out, cnts_out,
                     src_idx, tgt_idx, rows0, rows1, ones_v, acc, cnt,
                     gsem0, gsem1):
    c = lax.axis_index("c")
    s = lax.axis_index("s")

    pltpu.sync_copy(zacc_hbm.at[pl.ds(s * RPT, RPT)], acc.at[pl.ds(s * RPT, RPT)])
    pltpu.sync_copy(zcnt_hbm.at[pl.ds(s * RPT, RPT)], cnt.at[pl.ds(s * RPT, RPT)])

    pltpu.sync_copy(esrc_hbm.at[s], src_idx)
    pltpu.sync_copy(etgt_hbm.at[s], tgt_idx)
    pltpu.sync_copy(ones_hbm, ones_v)

    plsc.subcore_barrier()

    def start_gather(i, rows, gsem):
        @pl.when(c == 0)
        def _():
            pltpu.async_copy(xlo_hbm.at[src_idx.at[i]], rows, gsem)

        @pl.when(c == 1)
        def _():
            pltpu.async_copy(xhi_hbm.at[src_idx.at[i]], rows, gsem)

    def finish_chunk(i, rows, gsem):
        pltpu.make_async_copy(xlo_hbm.at[src_idx.at[i]], rows, gsem).wait()
        pltpu.sync_copy(rows, acc.at[tgt_idx.at[i]], add=True)

        @pl.when(lax.rem(i, 2) == c)
        def _():
            pltpu.sync_copy(ones_v, cnt.at[tgt_idx.at[i]], add=True)

    start_gather(0, rows0, gsem0)
    start_gather(1, rows1, gsem1)

    def chunk_pair(j, carry):
        i = 2 * j
        finish_chunk(i, rows0, gsem0)

        @pl.when(i + 2 < NCH)
        def _():
            start_gather(i + 2, rows0, gsem0)

        finish_chunk(i + 1, rows1, gsem1)

        @pl.when(i + 3 < NCH)
        def _():
            start_gather(i + 3, rows1, gsem1)

        return carry

    lax.fori_loop(0, NCH // 2, chunk_pair, 0)

    plsc.subcore_barrier()

    pltpu.sync_copy(acc.at[pl.ds(s * RPT, RPT)], sums_out.at[c, pl.ds(s * RPT, RPT)])
    pltpu.sync_copy(cnt.at[pl.ds(s * RPT, RPT)], cnts_out.at[c, pl.ds(s * RPT, RPT)])


_sc_scatter = functools.partial(
    pl.kernel,
    out_type=(
        jax.ShapeDtypeStruct((NC, N_PAD, HD), jnp.float32),
        jax.ShapeDtypeStruct((NC, N_PAD, CW), jnp.float32),
    ),
    mesh=plsc.VectorSubcoreMesh(
        core_axis_name="c", subcore_axis_name="s",
        num_cores=NC, num_subcores=NS),
    compiler_params=pltpu.CompilerParams(use_tc_tiling_on_sc=False),
    scratch_types=[
        pltpu.VMEM((NCH, B), jnp.int32),
        pltpu.VMEM((NCH, B), jnp.int32),
        pltpu.VMEM((B, HD), jnp.float32),
        pltpu.VMEM((B, HD), jnp.float32),
        pltpu.VMEM((B, CW), jnp.float32),
        pltpu.VMEM_SHARED((N_PAD, HD), jnp.float32),
        pltpu.VMEM_SHARED((N_PAD, CW), jnp.float32),
        pltpu.SemaphoreType.DMA,
        pltpu.SemaphoreType.DMA,
    ],
)(_sc_scatter_body)


def _combine_body(x_ref, s_ref, cp_ref, ws_ref, wt_ref, o_ref):
    cnt = cp_ref[0, :, 0:1] + cp_ref[1, :, 0:1]
    sums = jnp.concatenate([s_ref[0], s_ref[1]], axis=1)
    mean = sums / jnp.maximum(cnt, 1.0)
    o_ref[...] = (
        jnp.dot(x_ref[...], wt_ref[...], preferred_element_type=jnp.float32)
        + jnp.dot(mean, ws_ref[...], preferred_element_type=jnp.float32)
    )


BN = 2000


def _combine(x, sums, cnts_p, W_src, W_tgt):
    grid = (N // BN,)
    return pl.pallas_call(
        _combine_body,
        grid=grid,
        in_specs=[
            pl.BlockSpec((BN, D), lambda i: (i, 0)),
            pl.BlockSpec((2, BN, HD), lambda i: (0, i, 0)),
            pl.BlockSpec((2, BN, CW), lambda i: (0, i, 0)),
            pl.BlockSpec((D, OUT), lambda i: (0, 0)),
            pl.BlockSpec((D, OUT), lambda i: (0, 0)),
        ],
        out_specs=pl.BlockSpec((BN, OUT), lambda i: (i, 0)),
        out_shape=jax.ShapeDtypeStruct((N, OUT), jnp.float32),
    )(x, sums, cnts_p, W_src, W_tgt)


def kernel(x, edge_index, W_src, W_tgt):
    xlo = x[:, :HD]
    xhi = x[:, HD:]
    esrc = edge_index[0].reshape(NS, NCH, B)
    etgt = edge_index[1].reshape(NS, NCH, B)
    zacc = jnp.zeros((N_PAD, HD), jnp.float32)
    zcnt = jnp.zeros((N_PAD, CW), jnp.float32)
    ones = jnp.ones((B, CW), jnp.float32)
    sums, cnts_p = _sc_scatter(xlo, xhi, esrc, etgt, zacc, zcnt, ones)
    return _combine(x, sums, cnts_p, W_src, W_tgt)


# speedup vs baseline: 2.1502x; 1.0752x over previous
import functools

import jax
import jax.numpy as jnp
from jax import lax
from jax.experimental import pallas as pl
from jax.experimental.pallas import tpu as pltpu
from jax.experimental.pallas import tpu_sc as plsc

N = 10000
E = 320000
D = 128
OUT = 128

NC = 2
NS = 16
HD = D // NC
EPT = E // NS
B = 250
NCH = EPT // B
N_PAD = 10240
RPT = N_PAD // NS
CW = 8


def _sc_scatter_body(x2_hbm, esrc_hbm, etgt_hbm,
                     zacc_hbm, zcnt_hbm, ones_hbm,
                     sums_out, cnts_out,
                     src_idx, tgt_idx, rows0, rows1, ones_v, acc, cnt,
                     gsem0, gsem1):
    c = lax.axis_index("c")
    s = lax.axis_index("s")

    pltpu.sync_copy(zacc_hbm.at[pl.ds(s * RPT, RPT)], acc.at[pl.ds(s * RPT, RPT)])
    pltpu.sync_copy(zcnt_hbm.at[pl.ds(s * RPT, RPT)], cnt.at[pl.ds(s * RPT, RPT)])

    pltpu.sync_copy(esrc_hbm.at[c, s], src_idx)
    pltpu.sync_copy(etgt_hbm.at[s], tgt_idx)
    pltpu.sync_copy(ones_hbm, ones_v)

    plsc.subcore_barrier()

    def start_gather(i, rows, gsem):
        pltpu.async_copy(x2_hbm.at[src_idx.at[i]], rows, gsem)

    def finish_chunk(i, rows, gsem):
        pltpu.make_async_copy(x2_hbm.at[src_idx.at[i]], rows, gsem).wait()
        pltpu.sync_copy(rows, acc.at[tgt_idx.at[i]], add=True)

        @pl.when(lax.rem(i, 2) == c)
        def _():
            pltpu.sync_copy(ones_v, cnt.at[tgt_idx.at[i]], add=True)

    start_gather(0, rows0, gsem0)
    start_gather(1, rows1, gsem1)

    def chunk_pair(j, carry):
        i = 2 * j
        finish_chunk(i, rows0, gsem0)

        @pl.when(i + 2 < NCH)
        def _():
            start_gather(i + 2, rows0, gsem0)

        finish_chunk(i + 1, rows1, gsem1)

        @pl.when(i + 3 < NCH)
        def _():
            start_gather(i + 3, rows1, gsem1)

        return carry

    lax.fori_loop(0, NCH // 2, chunk_pair, 0)

    plsc.subcore_barrier()

    pltpu.sync_copy(acc.at[pl.ds(s * RPT, RPT)], sums_out.at[c, pl.ds(s * RPT, RPT)])
    pltpu.sync_copy(cnt.at[pl.ds(s * RPT, RPT)], cnts_out.at[c, pl.ds(s * RPT, RPT)])


_sc_scatter = functools.partial(
    pl.kernel,
    out_type=(
        jax.ShapeDtypeStruct((NC, N_PAD, HD), jnp.float32),
        jax.ShapeDtypeStruct((NC, N_PAD, CW), jnp.float32),
    ),
    mesh=plsc.VectorSubcoreMesh(
        core_axis_name="c", subcore_axis_name="s",
        num_cores=NC, num_subcores=NS),
    compiler_params=pltpu.CompilerParams(use_tc_tiling_on_sc=False),
    scratch_types=[
        pltpu.VMEM((NCH, B), jnp.int32),
        pltpu.VMEM((NCH, B), jnp.int32),
        pltpu.VMEM((B, HD), jnp.float32),
        pltpu.VMEM((B, HD), jnp.float32),
        pltpu.VMEM((B, CW), jnp.float32),
        pltpu.VMEM_SHARED((N_PAD, HD), jnp.float32),
        pltpu.VMEM_SHARED((N_PAD, CW), jnp.float32),
        pltpu.SemaphoreType.DMA,
        pltpu.SemaphoreType.DMA,
    ],
)(_sc_scatter_body)


def _combine_body(x_ref, s_ref, cp_ref, ws_ref, wt_ref, o_ref):
    cnt = cp_ref[0, :, 0:1] + cp_ref[1, :, 0:1]
    sums = jnp.concatenate([s_ref[0], s_ref[1]], axis=1)
    mean = sums / jnp.maximum(cnt, 1.0)
    o_ref[...] = (
        jnp.dot(x_ref[...], wt_ref[...], preferred_element_type=jnp.float32)
        + jnp.dot(mean, ws_ref[...], preferred_element_type=jnp.float32)
    )


BN = 2000


def _combine(x, sums, cnts_p, W_src, W_tgt):
    grid = (N // BN,)
    return pl.pallas_call(
        _combine_body,
        grid=grid,
        in_specs=[
            pl.BlockSpec((BN, D), lambda i: (i, 0)),
            pl.BlockSpec((2, BN, HD), lambda i: (0, i, 0)),
            pl.BlockSpec((2, BN, CW), lambda i: (0, i, 0)),
            pl.BlockSpec((D, OUT), lambda i: (0, 0)),
            pl.BlockSpec((D, OUT), lambda i: (0, 0)),
        ],
        out_specs=pl.BlockSpec((BN, OUT), lambda i: (i, 0)),
        out_shape=jax.ShapeDtypeStruct((N, OUT), jnp.float32),
    )(x, sums, cnts_p, W_src, W_tgt)


def kernel(x, edge_index, W_src, W_tgt):
    x2 = x.reshape(NC * N, HD)
    src = edge_index[0]
    esrc2 = jnp.stack([2 * src, 2 * src + 1]).reshape(NC, NS, NCH, B)
    etgt = edge_index[1].reshape(NS, NCH, B)
    zacc = jnp.zeros((N_PAD, HD), jnp.float32)
    zcnt = jnp.zeros((N_PAD, CW), jnp.float32)
    ones = jnp.ones((B, CW), jnp.float32)
    sums, cnts_p = _sc_scatter(x2, esrc2, etgt, zacc, zcnt, ones)
    return _combine(x, sums, cnts_p, W_src, W_tgt)
